# Initial kernel scaffold; baseline (speedup 1.0000x reference)
import functools

import jax
import jax.numpy as jnp
from jax import lax
from jax.experimental import pallas as pl
from jax.experimental.pallas import tpu as pltpu
from jax.experimental.pallas import tpu_sc as plsc

_NC = 2
_NS = 16
_NW = _NC * _NS
_CH = 128


def _zero_body(o_ref):
    o_ref[...] = jnp.zeros_like(o_ref)


def _tc_zeros(M, D, block_rows):
    return pl.pallas_call(
        _zero_body,
        grid=(M // block_rows,),
        out_specs=pl.BlockSpec((block_rows, D), lambda i: (i, 0)),
        out_shape=jax.ShapeDtypeStruct((M, D), jnp.float32),
    )()


def _sc_scatter(upd, idx2d, out_ref):
    N, D = upd.shape
    per_w = N // _NW
    n_ch = per_w // _CH

    mesh = plsc.VectorSubcoreMesh(core_axis_name="c", subcore_axis_name="s")

    @functools.partial(
        pl.kernel,
        mesh=mesh,
        scratch_types=[
            pltpu.VMEM((n_ch, _CH), jnp.int32),
            pltpu.VMEM((per_w, D), jnp.float32),
            pltpu.SemaphoreType.DMA,
        ],
    )
    def scatter_kernel(upd_hbm, idx_hbm, z_hbm, idx_v, rows_v, sem):
        wid = lax.axis_index("s") * _NC + lax.axis_index("c")
        cp_i = pltpu.async_copy(idx_hbm.at[pl.ds(wid * n_ch, n_ch)], idx_v, sem)
        cp_r = pltpu.async_copy(upd_hbm.at[pl.ds(wid * per_w, per_w)], rows_v, sem)
        cp_i.wait()
        cp_r.wait()
        for j in range(n_ch):
            pltpu.sync_copy(
                rows_v.at[pl.ds(j * _CH, _CH)], z_hbm.at[idx_v.at[j]]
            )

    scatter_kernel(upd, idx2d, out_ref)


def kernel(target_g, original_level_h, original_level_idx):
    M = target_g.shape[0]
    N, D = original_level_h.shape
    idx = original_level_idx.astype(jnp.int32)

    pos = jnp.arange(N, dtype=jnp.int32)
    tick = jnp.zeros((M,), jnp.int32).at[idx].max(pos + 1)
    winner = tick[idx] - 1
    upd = original_level_h[winner]

    idx2d = idx.reshape(N // _CH, _CH)

    zeros = _tc_zeros(M, D, 8000)
    out_ref = jax.new_ref(zeros)
    _sc_scatter(upd, idx2d, out_ref)
    return out_ref[...]


# speedup vs baseline: 2.1925x; 2.1925x over previous
import functools

import jax
import jax.numpy as jnp
from jax import lax
from jax.experimental import pallas as pl
from jax.experimental.pallas import tpu as pltpu
from jax.experimental.pallas import tpu_sc as plsc

_NC = 2
_NS = 16
_CH = 128
_ZR = 625


def _sc_fill_scatter(idx3d, upd3d, M):
    NC_, N, D = upd3d.shape
    per_w = N // _NS
    n_ch = per_w // _CH
    half = M // _NC
    rows_w = half // _NS
    n_z = rows_w // _ZR

    mesh = plsc.VectorSubcoreMesh(
        core_axis_name="c", subcore_axis_name="s",
        num_cores=_NC, num_subcores=_NS,
    )

    @functools.partial(
        pl.kernel,
        out_type=jax.ShapeDtypeStruct((M, D), jnp.float32),
        mesh=mesh,
        compiler_params=pltpu.CompilerParams(use_tc_tiling_on_sc=False),
        scratch_types=[
            pltpu.VMEM((_ZR, D), jnp.float32),
            pltpu.VMEM((n_ch, _CH), jnp.int32),
            pltpu.VMEM((per_w, D), jnp.float32),
            pltpu.SemaphoreType.DMA,
            pltpu.SemaphoreType.DMA,
        ],
    )
    def fill_scatter(idx_hbm, upd_hbm, out_hbm, zbuf, idxv, updv, zsem, lsem):
        c = lax.axis_index("c")
        s = lax.axis_index("s")

        @pl.loop(0, _ZR)
        def _(r):
            for t in range(D // 16):
                zbuf[r, pl.ds(t * 16, 16)] = jnp.zeros((16,), jnp.float32)

        base = c * half + s * rows_w
        copies = [
            pltpu.async_copy(
                zbuf, out_hbm.at[pl.ds(base + k * _ZR, _ZR), :], zsem
            )
            for k in range(n_z)
        ]

        cp_i = pltpu.async_copy(idx_hbm.at[c, pl.ds(s * n_ch, n_ch)], idxv, lsem)
        cp_u = pltpu.async_copy(upd_hbm.at[c, pl.ds(s * per_w, per_w)], updv, lsem)
        cp_i.wait()
        cp_u.wait()
        for cp in copies:
            cp.wait()

        plsc.subcore_barrier()

        for j in range(n_ch):
            pltpu.sync_copy(
                updv.at[pl.ds(j * _CH, _CH)], out_hbm.at[idxv.at[j]]
            )

    return fill_scatter(idx3d, upd3d)


def kernel(target_g, original_level_h, original_level_idx):
    M = target_g.shape[0]
    N, D = original_level_h.shape
    idx = original_level_idx.astype(jnp.int32)

    pos = jnp.arange(N, dtype=jnp.int32)
    tick = jnp.zeros((M,), jnp.int32).at[idx].max(pos + 1)
    winner = tick[idx] - 1
    upd = original_level_h[winner]

    half = M // _NC
    idx_c, upd_c = [], []
    for c in range(_NC):
        in_half = (idx >= c * half) & (idx < (c + 1) * half)
        first = jnp.argmax(in_half)
        has = in_half[first]
        pad_idx = jnp.where(has, idx[first], c * half)
        pad_val = jnp.where(has, upd[first], jnp.zeros((D,), upd.dtype))
        idx_c.append(jnp.where(in_half, idx, pad_idx))
        upd_c.append(jnp.where(in_half[:, None], upd, pad_val[None, :]))
    idx3d = jnp.stack(idx_c).reshape(_NC, N // _CH, _CH)
    upd3d = jnp.stack(upd_c)

    return _sc_fill_scatter(idx3d, upd3d, M)
